# baseline (speedup 1.0000x reference)
import functools

import jax
import jax.numpy as jnp
from jax import lax
from jax.experimental import pallas as pl
from jax.experimental.pallas import tpu as pltpu
from jax.experimental.pallas import tpu_sc as plsc

D = 2048
P = 16
SUB = 256
C = 64
PC = P * C
BETA = 0.1
NEG = -3.4e38
BT = 512
BTC = 2048


def _fold_kernel(uw_ref, cdw_ref, rd_ref, bias_ref, tr_ref):
    p = pl.program_id(0)
    uw = uw_ref[0]
    cdw = cdw_ref[0]
    t = jax.lax.dot_general(
        cdw, uw, (((0,), (1,)), ((), ())), preferred_element_type=jnp.float32)
    sel = (jax.lax.broadcasted_iota(jnp.int32, (P, 1), 0) == p).astype(jnp.float32)
    rd_row = jax.lax.dot_general(
        sel, rd_ref[...], (((0,), (1,)), ((), ())), preferred_element_type=jnp.float32)
    tr_ref[...] = t + BETA * rd_row + 0.5 * bias_ref[...]


def _select_kernel(x_ref, rew_ref, reb_ref, ceb_ref, dw_ref, cew_ref,
                   bias_ref, f1_ref, f2_ref):
    xb = x_ref[...] - bias_ref[...]
    plog = jax.lax.dot_general(
        xb, rew_ref[...], (((1,), (1,)), ((), ())),
        preferred_element_type=jnp.float32) + reb_ref[...]
    iota_p = jax.lax.broadcasted_iota(jnp.int32, plog.shape, 1)
    m1 = jnp.max(plog, axis=1, keepdims=True)
    i1 = jnp.min(jnp.where(plog == m1, iota_p, P), axis=1, keepdims=True)
    plog2 = jnp.where(iota_p == i1, NEG, plog)
    m2 = jnp.max(plog2, axis=1, keepdims=True)
    i2 = jnp.min(jnp.where(plog2 == m2, iota_p, P), axis=1, keepdims=True)

    z = jax.lax.dot_general(
        xb, dw_ref[...], (((1,), (1,)), ((), ())),
        preferred_element_type=jnp.float32)
    c1 = jnp.zeros_like(i1)
    c2 = jnp.zeros_like(i2)
    iota_c = jax.lax.broadcasted_iota(jnp.int32, (z.shape[0], C), 1)
    for p in range(P):
        clog = jax.lax.dot_general(
            z[:, p * SUB:(p + 1) * SUB], cew_ref[p],
            (((1,), (1,)), ((), ())),
            preferred_element_type=jnp.float32) + ceb_ref[:, p * C:(p + 1) * C]
        mx = jnp.max(clog, axis=1, keepdims=True)
        cid = jnp.min(jnp.where(clog == mx, iota_c, C), axis=1, keepdims=True)
        c1 = jnp.where(i1 == p, cid, c1)
        c2 = jnp.where(i2 == p, cid, c2)
    f1_ref[...] = i1 * C + c1
    f2_ref[...] = i2 * C + c2


def _combine_kernel(f1_ref, f2_ref, tr_ref, out_ref):
    f1 = f1_ref[...]
    f2 = f2_ref[...]
    iota_pc = jax.lax.broadcasted_iota(jnp.int32, (f1.shape[0], PC), 1)
    onehot = ((iota_pc == f1) | (iota_pc == f2)).astype(jnp.float32)
    out_ref[...] = jax.lax.dot_general(
        onehot, tr_ref[...], (((1,), (0,)), ((), ())),
        preferred_element_type=jnp.float32)


_SC_NT = 8


def _sc_combine_body(tr_hbm, f1_hbm, f2_hbm, out_hbm,
                     idx1_v, idx2_v,
                     b1a, b2a, oa, b1b, b2b, ob,
                     s1a, s2a, soa, s1b, s2b, sob):
    info = plsc.get_sparse_core_info()
    nw = info.num_cores * info.num_subcores
    bpw = out_hbm.shape[0] // nw
    wid = lax.axis_index("s") * info.num_cores + lax.axis_index("c")
    base = wid * bpw
    nsup = bpw // (2 * _SC_NT)
    pltpu.sync_copy(f1_hbm.at[pl.ds(base, bpw)], idx1_v)
    pltpu.sync_copy(f2_hbm.at[pl.ds(base, bpw)], idx2_v)

    def fire(c, b1, b2, s1, s2):
        t0 = c * _SC_NT
        pltpu.async_copy(tr_hbm.at[idx1_v.at[pl.ds(t0, _SC_NT)]], b1, s1)
        pltpu.async_copy(tr_hbm.at[idx2_v.at[pl.ds(t0, _SC_NT)]], b2, s2)

    def wait_gather(buf, sem):
        pltpu.make_async_copy(tr_hbm.at[pl.ds(0, _SC_NT)], buf, sem).wait()

    def wait_write(obuf, sem):
        pltpu.make_async_copy(obuf, out_hbm.at[pl.ds(base, _SC_NT)], sem).wait()

    def add_and_emit(c, b1, b2, obuf, sem_o):
        def vec(j, carry):
            sl = pl.ds(j * 16, 16)
            for i in range(_SC_NT):
                obuf[i, sl] = b1[i, sl] + b2[i, sl]
            return carry
        lax.fori_loop(0, D // 16, vec, 0)
        pltpu.async_copy(obuf, out_hbm.at[pl.ds(base + c * _SC_NT, _SC_NT)], sem_o)

    fire(0, b1a, b2a, s1a, s2a)
    fire(1, b1b, b2b, s1b, s2b)

    def superstep(s, carry):
        c0 = s * 2
        wait_gather(b1a, s1a)
        wait_gather(b2a, s2a)

        @pl.when(s > 0)
        def _():
            wait_write(oa, soa)
        add_and_emit(c0, b1a, b2a, oa, soa)

        @pl.when(s < nsup - 1)
        def _():
            fire(c0 + 2, b1a, b2a, s1a, s2a)
        wait_gather(b1b, s1b)
        wait_gather(b2b, s2b)

        @pl.when(s > 0)
        def _():
            wait_write(ob, sob)
        add_and_emit(c0 + 1, b1b, b2b, ob, sob)

        @pl.when(s < nsup - 1)
        def _():
            fire(c0 + 3, b1b, b2b, s1b, s2b)
        return carry

    lax.fori_loop(0, nsup, superstep, 0)
    wait_write(oa, soa)
    wait_write(ob, sob)


def _sc_combine(trows, f1, f2, B):
    mesh = plsc.VectorSubcoreMesh(core_axis_name="c", subcore_axis_name="s")
    info = plsc.get_sparse_core_info()
    nw = info.num_cores * info.num_subcores
    run = pl.kernel(
        _sc_combine_body,
        out_type=jax.ShapeDtypeStruct((B, D), jnp.float32),
        mesh=mesh,
        scratch_types=[
            pltpu.VMEM((B // nw,), jnp.int32),
            pltpu.VMEM((B // nw,), jnp.int32),
            pltpu.VMEM((_SC_NT, D), jnp.float32),
            pltpu.VMEM((_SC_NT, D), jnp.float32),
            pltpu.VMEM((_SC_NT, D), jnp.float32),
            pltpu.VMEM((_SC_NT, D), jnp.float32),
            pltpu.VMEM((_SC_NT, D), jnp.float32),
            pltpu.VMEM((_SC_NT, D), jnp.float32),
            pltpu.SemaphoreType.DMA,
            pltpu.SemaphoreType.DMA,
            pltpu.SemaphoreType.DMA,
            pltpu.SemaphoreType.DMA,
            pltpu.SemaphoreType.DMA,
            pltpu.SemaphoreType.DMA,
        ],
    )
    return run(trows, f1, f2)


def kernel(x, router_enc_w, router_enc_b, router_dec_w, decoder_bias,
           down_w, up_w, child_enc_w, child_enc_b, child_dec_w):
    B = x.shape[0]
    bias_row = decoder_bias.reshape(1, D)
    reb = router_enc_b.reshape(1, P)
    ceb = child_enc_b.reshape(1, PC)
    dw_flat = down_w.reshape(P * SUB, D)

    trows = pl.pallas_call(
        _fold_kernel,
        grid=(P,),
        in_specs=[
            pl.BlockSpec((1, D, SUB), lambda p: (p, 0, 0)),
            pl.BlockSpec((1, SUB, C), lambda p: (p, 0, 0)),
            pl.BlockSpec((D, P), lambda p: (0, 0)),
            pl.BlockSpec((1, D), lambda p: (0, 0)),
        ],
        out_specs=pl.BlockSpec((C, D), lambda p: (p, 0)),
        out_shape=jax.ShapeDtypeStruct((PC, D), jnp.float32),
        compiler_params=pltpu.CompilerParams(
            dimension_semantics=("parallel",)),
    )(up_w, child_dec_w, router_dec_w, bias_row)

    f1, f2 = pl.pallas_call(
        _select_kernel,
        grid=(B // BT,),
        in_specs=[
            pl.BlockSpec((BT, D), lambda i: (i, 0)),
            pl.BlockSpec((P, D), lambda i: (0, 0)),
            pl.BlockSpec((1, P), lambda i: (0, 0)),
            pl.BlockSpec((1, PC), lambda i: (0, 0)),
            pl.BlockSpec((P * SUB, D), lambda i: (0, 0)),
            pl.BlockSpec((P, C, SUB), lambda i: (0, 0, 0)),
            pl.BlockSpec((1, D), lambda i: (0, 0)),
        ],
        out_specs=[
            pl.BlockSpec((BT, 1), lambda i: (i, 0)),
            pl.BlockSpec((BT, 1), lambda i: (i, 0)),
        ],
        out_shape=[
            jax.ShapeDtypeStruct((B, 1), jnp.int32),
            jax.ShapeDtypeStruct((B, 1), jnp.int32),
        ],
        compiler_params=pltpu.CompilerParams(
            dimension_semantics=("arbitrary",)),
    )(x, router_enc_w, reb, ceb, dw_flat, child_enc_w, bias_row)

    Bh = B // 2
    out_sc = _sc_combine(trows,
                         lax.slice_in_dim(f1, Bh, B, axis=0).reshape(Bh),
                         lax.slice_in_dim(f2, Bh, B, axis=0).reshape(Bh), Bh)
    out_tc = pl.pallas_call(
        _combine_kernel,
        grid=(Bh // BTC,),
        in_specs=[
            pl.BlockSpec((BTC, 1), lambda i: (i, 0)),
            pl.BlockSpec((BTC, 1), lambda i: (i, 0)),
            pl.BlockSpec((PC, D), lambda i: (0, 0)),
        ],
        out_specs=pl.BlockSpec((BTC, D), lambda i: (i, 0)),
        out_shape=jax.ShapeDtypeStruct((Bh, D), jnp.float32),
        compiler_params=pltpu.CompilerParams(
            dimension_semantics=("arbitrary",)),
    )(lax.slice_in_dim(f1, 0, Bh, axis=0), lax.slice_in_dim(f2, 0, Bh, axis=0),
      trows)
    return jnp.concatenate([out_tc, out_sc], axis=0)


# speedup vs baseline: 1.0464x; 1.0464x over previous
import functools

import jax
import jax.numpy as jnp
from jax import lax
from jax.experimental import pallas as pl
from jax.experimental.pallas import tpu as pltpu
from jax.experimental.pallas import tpu_sc as plsc

D = 2048
P = 16
SUB = 256
C = 64
PC = P * C
BETA = 0.1
NEG = -3.4e38
BT = 512
BTC = 2048


def _fold_kernel(uw_ref, cdw_ref, rd_ref, bias_ref, tr_ref):
    p = pl.program_id(0)
    uw = uw_ref[0]
    cdw = cdw_ref[0]
    t = jax.lax.dot_general(
        cdw, uw, (((0,), (1,)), ((), ())), preferred_element_type=jnp.float32)
    sel = (jax.lax.broadcasted_iota(jnp.int32, (P, 1), 0) == p).astype(jnp.float32)
    rd_row = jax.lax.dot_general(
        sel, rd_ref[...], (((0,), (1,)), ((), ())), preferred_element_type=jnp.float32)
    tr_ref[...] = t + BETA * rd_row + 0.5 * bias_ref[...]


def _select_kernel(x_ref, rew_ref, reb_ref, ceb_ref, dw_ref, cew_ref,
                   bias_ref, f1_ref, f2_ref):
    xb = x_ref[...] - bias_ref[...]
    plog = jax.lax.dot_general(
        xb, rew_ref[...], (((1,), (1,)), ((), ())),
        preferred_element_type=jnp.float32) + reb_ref[...]
    iota_p = jax.lax.broadcasted_iota(jnp.int32, plog.shape, 1)
    m1 = jnp.max(plog, axis=1, keepdims=True)
    i1 = jnp.min(jnp.where(plog == m1, iota_p, P), axis=1, keepdims=True)
    plog2 = jnp.where(iota_p == i1, NEG, plog)
    m2 = jnp.max(plog2, axis=1, keepdims=True)
    i2 = jnp.min(jnp.where(plog2 == m2, iota_p, P), axis=1, keepdims=True)

    z = jax.lax.dot_general(
        xb, dw_ref[...], (((1,), (1,)), ((), ())),
        preferred_element_type=jnp.float32)
    c1 = jnp.zeros_like(i1)
    c2 = jnp.zeros_like(i2)
    iota_c = jax.lax.broadcasted_iota(jnp.int32, (z.shape[0], C), 1)
    for p in range(P):
        clog = jax.lax.dot_general(
            z[:, p * SUB:(p + 1) * SUB], cew_ref[p],
            (((1,), (1,)), ((), ())),
            preferred_element_type=jnp.float32) + ceb_ref[:, p * C:(p + 1) * C]
        mx = jnp.max(clog, axis=1, keepdims=True)
        cid = jnp.min(jnp.where(clog == mx, iota_c, C), axis=1, keepdims=True)
        c1 = jnp.where(i1 == p, cid, c1)
        c2 = jnp.where(i2 == p, cid, c2)
    f1_ref[...] = i1 * C + c1
    f2_ref[...] = i2 * C + c2


def _combine_kernel(f1_ref, f2_ref, tr_ref, out_ref):
    f1 = f1_ref[...]
    f2 = f2_ref[...]
    iota_pc = jax.lax.broadcasted_iota(jnp.int32, (f1.shape[0], PC), 1)
    onehot = ((iota_pc == f1) | (iota_pc == f2)).astype(jnp.float32)
    out_ref[...] = jax.lax.dot_general(
        onehot, tr_ref[...], (((1,), (0,)), ((), ())),
        preferred_element_type=jnp.float32)


_SC_NT = 8


def _sc_combine_body(tr_hbm, f1_hbm, f2_hbm, out_hbm,
                     idx1_v, idx2_v,
                     b1a, b2a, oa, b1b, b2b, ob,
                     s1a, s2a, soa, s1b, s2b, sob):
    info = plsc.get_sparse_core_info()
    nw = info.num_cores * info.num_subcores
    bpw = out_hbm.shape[0] // nw
    wid = lax.axis_index("s") * info.num_cores + lax.axis_index("c")
    base = wid * bpw
    nsup = bpw // (2 * _SC_NT)
    pltpu.sync_copy(f1_hbm.at[pl.ds(base, bpw)], idx1_v)
    pltpu.sync_copy(f2_hbm.at[pl.ds(base, bpw)], idx2_v)

    def fire(c, b1, b2, s1, s2):
        t0 = c * _SC_NT
        pltpu.async_copy(tr_hbm.at[idx1_v.at[pl.ds(t0, _SC_NT)]], b1, s1)
        pltpu.async_copy(tr_hbm.at[idx2_v.at[pl.ds(t0, _SC_NT)]], b2, s2)

    def wait_gather(buf, sem):
        pltpu.make_async_copy(tr_hbm.at[pl.ds(0, _SC_NT)], buf, sem).wait()

    def wait_write(obuf, sem):
        pltpu.make_async_copy(obuf, out_hbm.at[pl.ds(base, _SC_NT)], sem).wait()

    def add_and_emit(c, b1, b2, obuf, sem_o):
        def vec(j, carry):
            sl = pl.ds(j * 16, 16)
            for i in range(_SC_NT):
                obuf[i, sl] = b1[i, sl] + b2[i, sl]
            return carry
        lax.fori_loop(0, D // 16, vec, 0)
        pltpu.async_copy(obuf, out_hbm.at[pl.ds(base + c * _SC_NT, _SC_NT)], sem_o)

    fire(0, b1a, b2a, s1a, s2a)
    fire(1, b1b, b2b, s1b, s2b)

    def superstep(s, carry):
        c0 = s * 2
        wait_gather(b1a, s1a)
        wait_gather(b2a, s2a)

        @pl.when(s > 0)
        def _():
            wait_write(oa, soa)
        add_and_emit(c0, b1a, b2a, oa, soa)

        @pl.when(s < nsup - 1)
        def _():
            fire(c0 + 2, b1a, b2a, s1a, s2a)
        wait_gather(b1b, s1b)
        wait_gather(b2b, s2b)

        @pl.when(s > 0)
        def _():
            wait_write(ob, sob)
        add_and_emit(c0 + 1, b1b, b2b, ob, sob)

        @pl.when(s < nsup - 1)
        def _():
            fire(c0 + 3, b1b, b2b, s1b, s2b)
        return carry

    lax.fori_loop(0, nsup, superstep, 0)
    wait_write(oa, soa)
    wait_write(ob, sob)


def _sc_combine(trows, f1, f2, B):
    mesh = plsc.VectorSubcoreMesh(core_axis_name="c", subcore_axis_name="s")
    info = plsc.get_sparse_core_info()
    nw = info.num_cores * info.num_subcores
    run = pl.kernel(
        _sc_combine_body,
        out_type=jax.ShapeDtypeStruct((B, D), jnp.float32),
        mesh=mesh,
        scratch_types=[
            pltpu.VMEM((B // nw,), jnp.int32),
            pltpu.VMEM((B // nw,), jnp.int32),
            pltpu.VMEM((_SC_NT, D), jnp.float32),
            pltpu.VMEM((_SC_NT, D), jnp.float32),
            pltpu.VMEM((_SC_NT, D), jnp.float32),
            pltpu.VMEM((_SC_NT, D), jnp.float32),
            pltpu.VMEM((_SC_NT, D), jnp.float32),
            pltpu.VMEM((_SC_NT, D), jnp.float32),
            pltpu.SemaphoreType.DMA,
            pltpu.SemaphoreType.DMA,
            pltpu.SemaphoreType.DMA,
            pltpu.SemaphoreType.DMA,
            pltpu.SemaphoreType.DMA,
            pltpu.SemaphoreType.DMA,
        ],
    )
    return run(trows, f1, f2)


def kernel(x, router_enc_w, router_enc_b, router_dec_w, decoder_bias,
           down_w, up_w, child_enc_w, child_enc_b, child_dec_w):
    B = x.shape[0]
    bias_row = decoder_bias.reshape(1, D)
    reb = router_enc_b.reshape(1, P)
    ceb = child_enc_b.reshape(1, PC)
    dw_flat = down_w.reshape(P * SUB, D)

    trows = pl.pallas_call(
        _fold_kernel,
        grid=(P,),
        in_specs=[
            pl.BlockSpec((1, D, SUB), lambda p: (p, 0, 0)),
            pl.BlockSpec((1, SUB, C), lambda p: (p, 0, 0)),
            pl.BlockSpec((D, P), lambda p: (0, 0)),
            pl.BlockSpec((1, D), lambda p: (0, 0)),
        ],
        out_specs=pl.BlockSpec((C, D), lambda p: (p, 0)),
        out_shape=jax.ShapeDtypeStruct((PC, D), jnp.float32),
        compiler_params=pltpu.CompilerParams(
            dimension_semantics=("parallel",)),
    )(up_w, child_dec_w, router_dec_w, bias_row)

    f1, f2 = pl.pallas_call(
        _select_kernel,
        grid=(B // BT,),
        in_specs=[
            pl.BlockSpec((BT, D), lambda i: (i, 0)),
            pl.BlockSpec((P, D), lambda i: (0, 0)),
            pl.BlockSpec((1, P), lambda i: (0, 0)),
            pl.BlockSpec((1, PC), lambda i: (0, 0)),
            pl.BlockSpec((P * SUB, D), lambda i: (0, 0)),
            pl.BlockSpec((P, C, SUB), lambda i: (0, 0, 0)),
            pl.BlockSpec((1, D), lambda i: (0, 0)),
        ],
        out_specs=[
            pl.BlockSpec((BT, 1), lambda i: (i, 0)),
            pl.BlockSpec((BT, 1), lambda i: (i, 0)),
        ],
        out_shape=[
            jax.ShapeDtypeStruct((B, 1), jnp.int32),
            jax.ShapeDtypeStruct((B, 1), jnp.int32),
        ],
        compiler_params=pltpu.CompilerParams(
            dimension_semantics=("arbitrary",)),
    )(x, router_enc_w, reb, ceb, dw_flat, child_enc_w, bias_row)

    return _sc_combine(trows, f1.reshape(B), f2.reshape(B), B)


# speedup vs baseline: 1.0472x; 1.0008x over previous
import jax
import jax.numpy as jnp
from jax import lax
from jax.experimental import pallas as pl
from jax.experimental.pallas import tpu as pltpu
from jax.experimental.pallas import tpu_sc as plsc

D = 2048
P = 16
SUB = 256
C = 64
PC = P * C
BETA = 0.1
NEG = -3.4e38
BT = 512


def _fold_kernel(uw_ref, cdw_ref, rd_ref, bias_ref, tr_ref):
    p = pl.program_id(0)
    uw = uw_ref[0]
    cdw = cdw_ref[0]
    t = jax.lax.dot_general(
        cdw, uw, (((0,), (1,)), ((), ())), preferred_element_type=jnp.float32)
    sel = (jax.lax.broadcasted_iota(jnp.int32, (P, 1), 0) == p).astype(jnp.float32)
    rd_row = jax.lax.dot_general(
        sel, rd_ref[...], (((0,), (1,)), ((), ())), preferred_element_type=jnp.float32)
    tr_ref[...] = t + BETA * rd_row + 0.5 * bias_ref[...]


def _select_kernel(x_ref, rew_ref, reb_ref, ceb_ref, dw_ref, cew_ref,
                   bias_ref, f1_ref, f2_ref):
    xb = x_ref[...] - bias_ref[...]
    plog = jax.lax.dot_general(
        xb, rew_ref[...], (((1,), (1,)), ((), ())),
        preferred_element_type=jnp.float32) + reb_ref[...]
    iota_p = jax.lax.broadcasted_iota(jnp.int32, plog.shape, 1)
    m1 = jnp.max(plog, axis=1, keepdims=True)
    i1 = jnp.min(jnp.where(plog == m1, iota_p, P), axis=1, keepdims=True)
    plog2 = jnp.where(iota_p == i1, NEG, plog)
    m2 = jnp.max(plog2, axis=1, keepdims=True)
    i2 = jnp.min(jnp.where(plog2 == m2, iota_p, P), axis=1, keepdims=True)

    z = jax.lax.dot_general(
        xb, dw_ref[...], (((1,), (1,)), ((), ())),
        preferred_element_type=jnp.float32)
    c1 = jnp.zeros_like(i1)
    c2 = jnp.zeros_like(i2)
    iota_c = jax.lax.broadcasted_iota(jnp.int32, (z.shape[0], C), 1)
    for p in range(P):
        clog = jax.lax.dot_general(
            z[:, p * SUB:(p + 1) * SUB], cew_ref[p],
            (((1,), (1,)), ((), ())),
            preferred_element_type=jnp.float32) + ceb_ref[:, p * C:(p + 1) * C]
        mx = jnp.max(clog, axis=1, keepdims=True)
        cid = jnp.min(jnp.where(clog == mx, iota_c, C), axis=1, keepdims=True)
        c1 = jnp.where(i1 == p, cid, c1)
        c2 = jnp.where(i2 == p, cid, c2)
    f1_ref[...] = i1 * C + c1
    f2_ref[...] = i2 * C + c2


_SC_NT = 8


def _sc_combine_body(tr_hbm, f1_hbm, f2_hbm, out_hbm,
                     idx1_v, idx2_v,
                     b1a, b2a, oa, b1b, b2b, ob,
                     s1a, s2a, soa, s1b, s2b, sob):
    info = plsc.get_sparse_core_info()
    nw = info.num_cores * info.num_subcores
    bpw = out_hbm.shape[0] // nw
    wid = lax.axis_index("s") * info.num_cores + lax.axis_index("c")
    base = wid * bpw
    nsup = bpw // (2 * _SC_NT)
    pltpu.sync_copy(f1_hbm.at[pl.ds(base, bpw)], idx1_v)
    pltpu.sync_copy(f2_hbm.at[pl.ds(base, bpw)], idx2_v)

    def fire(c, b1, b2, s1, s2):
        t0 = c * _SC_NT
        pltpu.async_copy(tr_hbm.at[idx1_v.at[pl.ds(t0, _SC_NT)]], b1, s1)
        pltpu.async_copy(tr_hbm.at[idx2_v.at[pl.ds(t0, _SC_NT)]], b2, s2)

    def wait_gather(buf, sem):
        pltpu.make_async_copy(tr_hbm.at[pl.ds(0, _SC_NT)], buf, sem).wait()

    def wait_write(obuf, sem):
        pltpu.make_async_copy(obuf, out_hbm.at[pl.ds(base, _SC_NT)], sem).wait()

    def add_and_emit(c, b1, b2, obuf, sem_o):
        def vec(j, carry):
            sl = pl.ds(j * 16, 16)
            for i in range(_SC_NT):
                obuf[i, sl] = b1[i, sl] + b2[i, sl]
            return carry
        lax.fori_loop(0, D // 16, vec, 0)
        pltpu.async_copy(obuf, out_hbm.at[pl.ds(base + c * _SC_NT, _SC_NT)], sem_o)

    fire(0, b1a, b2a, s1a, s2a)
    fire(1, b1b, b2b, s1b, s2b)

    def superstep(s, carry):
        c0 = s * 2
        wait_gather(b1a, s1a)
        wait_gather(b2a, s2a)

        @pl.when(s > 0)
        def _():
            wait_write(oa, soa)
        add_and_emit(c0, b1a, b2a, oa, soa)

        @pl.when(s < nsup - 1)
        def _():
            fire(c0 + 2, b1a, b2a, s1a, s2a)
        wait_gather(b1b, s1b)
        wait_gather(b2b, s2b)

        @pl.when(s > 0)
        def _():
            wait_write(ob, sob)
        add_and_emit(c0 + 1, b1b, b2b, ob, sob)

        @pl.when(s < nsup - 1)
        def _():
            fire(c0 + 3, b1b, b2b, s1b, s2b)
        return carry

    lax.fori_loop(0, nsup, superstep, 0)
    wait_write(oa, soa)
    wait_write(ob, sob)


def _sc_combine(trows, f1, f2, B):
    mesh = plsc.VectorSubcoreMesh(core_axis_name="c", subcore_axis_name="s")
    info = plsc.get_sparse_core_info()
    nw = info.num_cores * info.num_subcores
    run = pl.kernel(
        _sc_combine_body,
        out_type=jax.ShapeDtypeStruct((B, D), jnp.float32),
        mesh=mesh,
        scratch_types=[
            pltpu.VMEM((B // nw,), jnp.int32),
            pltpu.VMEM((B // nw,), jnp.int32),
            pltpu.VMEM((_SC_NT, D), jnp.float32),
            pltpu.VMEM((_SC_NT, D), jnp.float32),
            pltpu.VMEM((_SC_NT, D), jnp.float32),
            pltpu.VMEM((_SC_NT, D), jnp.float32),
            pltpu.VMEM((_SC_NT, D), jnp.float32),
            pltpu.VMEM((_SC_NT, D), jnp.float32),
            pltpu.SemaphoreType.DMA,
            pltpu.SemaphoreType.DMA,
            pltpu.SemaphoreType.DMA,
            pltpu.SemaphoreType.DMA,
            pltpu.SemaphoreType.DMA,
            pltpu.SemaphoreType.DMA,
        ],
    )
    return run(trows, f1, f2)


def kernel(x, router_enc_w, router_enc_b, router_dec_w, decoder_bias,
           down_w, up_w, child_enc_w, child_enc_b, child_dec_w):
    B = x.shape[0]
    bias_row = decoder_bias.reshape(1, D)
    reb = router_enc_b.reshape(1, P)
    ceb = child_enc_b.reshape(1, PC)
    dw_flat = down_w.reshape(P * SUB, D)

    trows = pl.pallas_call(
        _fold_kernel,
        grid=(P,),
        in_specs=[
            pl.BlockSpec((1, D, SUB), lambda p: (p, 0, 0)),
            pl.BlockSpec((1, SUB, C), lambda p: (p, 0, 0)),
            pl.BlockSpec((D, P), lambda p: (0, 0)),
            pl.BlockSpec((1, D), lambda p: (0, 0)),
        ],
        out_specs=pl.BlockSpec((C, D), lambda p: (p, 0)),
        out_shape=jax.ShapeDtypeStruct((PC, D), jnp.float32),
        compiler_params=pltpu.CompilerParams(
            dimension_semantics=("parallel",)),
    )(up_w, child_dec_w, router_dec_w, bias_row)

    f1, f2 = pl.pallas_call(
        _select_kernel,
        grid=(B // BT,),
        in_specs=[
            pl.BlockSpec((BT, D), lambda i: (i, 0)),
            pl.BlockSpec((P, D), lambda i: (0, 0)),
            pl.BlockSpec((1, P), lambda i: (0, 0)),
            pl.BlockSpec((1, PC), lambda i: (0, 0)),
            pl.BlockSpec((P * SUB, D), lambda i: (0, 0)),
            pl.BlockSpec((P, C, SUB), lambda i: (0, 0, 0)),
            pl.BlockSpec((1, D), lambda i: (0, 0)),
        ],
        out_specs=[
            pl.BlockSpec((BT, 1), lambda i: (i, 0)),
            pl.BlockSpec((BT, 1), lambda i: (i, 0)),
        ],
        out_shape=[
            jax.ShapeDtypeStruct((B, 1), jnp.int32),
            jax.ShapeDtypeStruct((B, 1), jnp.int32),
        ],
        compiler_params=pltpu.CompilerParams(
            dimension_semantics=("arbitrary",)),
    )(x, router_enc_w, reb, ceb, dw_flat, child_enc_w, bias_row)

    return _sc_combine(trows, f1.reshape(B), f2.reshape(B), B)
